# baseline (speedup 1.0000x reference)
import jax
import jax.numpy as jnp
from jax.experimental import pallas as pl
from jax.experimental.pallas import tpu as pltpu


def _layer1_kernel(x_ref, w_ref, adj_ref, o_ref, adjq_ref, h_ref):
    @pl.when(pl.program_id(0) == 0)
    def _():
        h_ref[...] = jnp.dot(
            x_ref[...], w_ref[...], preferred_element_type=jnp.float32
        ).astype(jnp.bfloat16)

    a = adj_ref[...]
    adjq_ref[...] = jnp.round((a - 0.5) * 254.0).astype(jnp.int8)
    o_ref[...] = jax.nn.relu(
        jnp.dot(
            a.astype(jnp.bfloat16), h_ref[...],
            preferred_element_type=jnp.float32,
        )
    ).astype(o_ref.dtype)


def _layer_q_kernel(x_ref, w_ref, adjq_ref, o_ref, h_ref, c_ref):
    @pl.when(pl.program_id(0) == 0)
    def _():
        h = jnp.dot(
            x_ref[...].astype(jnp.bfloat16),
            w_ref[...].astype(jnp.bfloat16),
            preferred_element_type=jnp.float32,
        )
        h_ref[...] = h.astype(jnp.bfloat16)
        c_ref[...] = 0.5 * jnp.sum(h, axis=0, keepdims=True)

    acc = jnp.dot(
        adjq_ref[...], h_ref[...], preferred_element_type=jnp.float32
    )
    o_ref[...] = jax.nn.relu(acc * (1.0 / 254.0) + c_ref[...]).astype(
        o_ref.dtype
    )


def _gcn_layer1(x, adj, w, blk):
    n, f = x.shape
    h = w.shape[1]
    return pl.pallas_call(
        _layer1_kernel,
        grid=(n // blk,),
        in_specs=[
            pl.BlockSpec((n, f), lambda i: (0, 0)),
            pl.BlockSpec((f, h), lambda i: (0, 0)),
            pl.BlockSpec((blk, n), lambda i: (i, 0)),
        ],
        out_specs=[
            pl.BlockSpec((blk, h), lambda i: (i, 0)),
            pl.BlockSpec((blk, n), lambda i: (i, 0)),
        ],
        out_shape=[
            jax.ShapeDtypeStruct((n, h), jnp.bfloat16),
            jax.ShapeDtypeStruct((n, n), jnp.int8),
        ],
        scratch_shapes=[pltpu.VMEM((n, h), jnp.bfloat16)],
    )(x, w, adj)


def _gcn_layer_q(x, adjq, w, blk, out_dtype):
    n, f = x.shape
    h = w.shape[1]
    return pl.pallas_call(
        _layer_q_kernel,
        grid=(n // blk,),
        in_specs=[
            pl.BlockSpec((n, f), lambda i: (0, 0)),
            pl.BlockSpec((f, h), lambda i: (0, 0)),
            pl.BlockSpec((blk, n), lambda i: (i, 0)),
        ],
        out_specs=pl.BlockSpec((blk, h), lambda i: (i, 0)),
        out_shape=jax.ShapeDtypeStruct((n, h), out_dtype),
        scratch_shapes=[
            pltpu.VMEM((n, h), jnp.bfloat16),
            pltpu.VMEM((1, h), jnp.float32),
        ],
    )(x, w, adjq)


def kernel(features, adj_matrix, W_in, W_h0, W_out):
    x, adjq = _gcn_layer1(features, adj_matrix, W_in, 400)
    x = _gcn_layer_q(x, adjq, W_h0, 2000, jnp.bfloat16)
    return _gcn_layer_q(x, adjq, W_out, 1000, jnp.float32)
